# Initial kernel scaffold; baseline (speedup 1.0000x reference)
import jax
import jax.numpy as jnp
from jax.experimental import pallas as pl


def kernel(x, segment_ids, neighbor_idx, W1, gamma1, beta1, W2, gamma2, beta2, W3, gamma3, beta3):
    raise NotImplementedError("write your pallas kernel here")


# speedup vs baseline: 5.7959x; 5.7959x over previous
import functools

import jax
import jax.numpy as jnp
from jax import lax
from jax.experimental import pallas as pl
from jax.experimental.pallas import tpu as pltpu
from jax.experimental.pallas import tpu_sc as plsc

_N = 8192
_B = 8
_INP = 512
_WIDTH = 128
_OUT = 512
_K = 27
_EPS = 1e-5
_HIGH = lax.Precision.HIGHEST


def _onehot_f32(seg):
    return (seg == lax.broadcasted_iota(jnp.int32, (1, _B), 1)).astype(jnp.float32)


def _seg_stats(rowsum, rowssq, onehot, c):
    dn = (((0,), (0,)), ((), ()))
    ssum = lax.dot_general(rowsum, onehot, dn, precision=_HIGH,
                           preferred_element_type=jnp.float32)
    ssq = lax.dot_general(rowssq, onehot, dn, precision=_HIGH,
                          preferred_element_type=jnp.float32)
    cnt = jnp.sum(onehot, axis=0, keepdims=True)
    denom = jnp.maximum(cnt * c, 1.0)
    mu = ssum / denom
    msq = ssq / denom
    var = jnp.maximum(msq - mu * mu, 0.0)
    inv = lax.rsqrt(var + _EPS)
    return mu, inv


def _per_row(onehot, v):
    return jnp.sum(onehot * v, axis=1, keepdims=True)


def _stage1_body(x_ref, seg_ref, w1_ref, g1_ref, b1_ref, out1_ref):
    y1 = jnp.dot(x_ref[...].astype(jnp.bfloat16),
                 w1_ref[...].astype(jnp.bfloat16),
                 preferred_element_type=jnp.float32)
    onehot = _onehot_f32(seg_ref[...])
    rowsum = jnp.sum(y1, axis=1, keepdims=True)
    rowssq = jnp.sum(y1 * y1, axis=1, keepdims=True)
    mu, inv = _seg_stats(rowsum, rowssq, onehot, float(_WIDTH))
    mu_r = _per_row(onehot, mu)
    inv_r = _per_row(onehot, inv)
    o = (y1 - mu_r) * inv_r * g1_ref[...] + b1_ref[...]
    out1_ref[...] = jnp.maximum(o, 0.0)


def _gather_taps(table, idx_flat):
    num_idx = idx_flat.shape[1]
    window = 128
    mesh = plsc.VectorSubcoreMesh(core_axis_name="c", subcore_axis_name="s")

    @functools.partial(
        pl.kernel,
        out_type=jax.ShapeDtypeStruct((num_idx, table.shape[1]), table.dtype),
        mesh=mesh)
    def _k(table_hbm, idx_hbm, out_hbm):
        def body(i_vmem, o_vmem):
            pltpu.sync_copy(table_hbm.at[i_vmem.at[0]], o_vmem)

        pltpu.emit_pipeline(
            body,
            grid=(num_idx // window,),
            in_specs=[pl.BlockSpec((1, window), index_map=lambda i: (0, i))],
            out_specs=[pl.BlockSpec((window, table.shape[1]),
                                    index_map=lambda i: (i, 0))],
            core_axis_name=("c", "s"),
            dimension_semantics=(pltpu.PARALLEL,),
        )(idx_hbm, out_hbm)

    return _k(table, idx_flat)


def _taps_body(nb_ref, w2_ref, seg_ref, g2_ref, b2_ref, out2_ref, acc_ref):
    k = pl.program_id(0)
    contrib = jnp.dot(nb_ref[...].astype(jnp.bfloat16),
                      w2_ref[0].astype(jnp.bfloat16),
                      preferred_element_type=jnp.float32)

    @pl.when(k == 0)
    def _():
        acc_ref[...] = contrib

    @pl.when(k > 0)
    def _():
        acc_ref[...] += contrib

    @pl.when(k == _K - 1)
    def _():
        y2 = acc_ref[...]
        onehot = _onehot_f32(seg_ref[...])
        rowsum = jnp.sum(y2, axis=1, keepdims=True)
        rowssq = jnp.sum(y2 * y2, axis=1, keepdims=True)
        mu, inv = _seg_stats(rowsum, rowssq, onehot, float(_WIDTH))
        mu_r = _per_row(onehot, mu)
        inv_r = _per_row(onehot, inv)
        o = (y2 - mu_r) * inv_r * g2_ref[...] + b2_ref[...]
        out2_ref[...] = jnp.maximum(o, 0.0)


def _expand_body(out2_ref, seg_ref, w3f_ref, w3c_ref, g3_ref, b3_ref, x_ref,
                 out_ref, stats_ref):
    i = pl.program_id(0)
    onehot = _onehot_f32(seg_ref[...])

    @pl.when(i == 0)
    def _():
        out2 = out2_ref[...]
        w3 = w3f_ref[...]
        dn_g = (((1,), (1,)), ((), ()))
        gram = lax.dot_general(w3, w3, dn_g, precision=_HIGH,
                               preferred_element_type=jnp.float32)
        h = jnp.dot(out2, gram, precision=_HIGH,
                    preferred_element_type=jnp.float32)
        rowssq = jnp.sum(h * out2, axis=1, keepdims=True)
        w3sum = jnp.sum(w3, axis=1, keepdims=True)
        rowsum = jnp.dot(out2, w3sum, precision=_HIGH,
                         preferred_element_type=jnp.float32)
        mu, inv = _seg_stats(rowsum, rowssq, onehot, float(_OUT))
        stats_ref[0:1, :] = mu
        stats_ref[1:2, :] = inv

    @pl.when(i > 0)
    def _():
        mu_r = _per_row(onehot, stats_ref[0:1, :])
        inv_r = _per_row(onehot, stats_ref[1:2, :])
        y3 = jnp.dot(out2_ref[...].astype(jnp.bfloat16),
                     w3c_ref[...].astype(jnp.bfloat16),
                     preferred_element_type=jnp.float32)
        o = (y3 - mu_r) * inv_r * g3_ref[...] + b3_ref[...] + x_ref[...]
        out_ref[...] = jnp.maximum(o, 0.0)


def kernel(x, segment_ids, neighbor_idx, W1, gamma1, beta1, W2, gamma2, beta2,
           W3, gamma3, beta3):
    seg2 = segment_ids.reshape(_N, 1)
    idx_flat = neighbor_idx.T.reshape(1, _K * _N)
    g1 = gamma1.reshape(1, _WIDTH)
    b1 = beta1.reshape(1, _WIDTH)
    g2 = gamma2.reshape(1, _WIDTH)
    b2 = beta2.reshape(1, _WIDTH)
    g3 = gamma3.reshape(1, _OUT)
    b3 = beta3.reshape(1, _OUT)

    out1 = pl.pallas_call(
        _stage1_body,
        out_shape=jax.ShapeDtypeStruct((_N, _WIDTH), jnp.float32),
    )(x, seg2, W1, g1, b1)

    nb = _gather_taps(out1, idx_flat)

    out2 = pl.pallas_call(
        _taps_body,
        grid=(_K,),
        in_specs=[
            pl.BlockSpec((_N, _WIDTH), lambda k: (k, 0)),
            pl.BlockSpec((1, _WIDTH, _WIDTH), lambda k: (k, 0, 0)),
            pl.BlockSpec((_N, 1), lambda k: (0, 0)),
            pl.BlockSpec((1, _WIDTH), lambda k: (0, 0)),
            pl.BlockSpec((1, _WIDTH), lambda k: (0, 0)),
        ],
        out_specs=pl.BlockSpec((_N, _WIDTH), lambda k: (0, 0)),
        out_shape=jax.ShapeDtypeStruct((_N, _WIDTH), jnp.float32),
        scratch_shapes=[pltpu.VMEM((_N, _WIDTH), jnp.float32)],
    )(nb, W2, seg2, g2, b2)

    nchunk = _OUT // 128
    out = pl.pallas_call(
        _expand_body,
        grid=(nchunk + 1,),
        in_specs=[
            pl.BlockSpec((_N, _WIDTH), lambda i: (0, 0)),
            pl.BlockSpec((_N, 1), lambda i: (0, 0)),
            pl.BlockSpec((_WIDTH, _OUT), lambda i: (0, 0)),
            pl.BlockSpec((_WIDTH, 128), lambda i: (0, jnp.maximum(i - 1, 0))),
            pl.BlockSpec((1, 128), lambda i: (0, jnp.maximum(i - 1, 0))),
            pl.BlockSpec((1, 128), lambda i: (0, jnp.maximum(i - 1, 0))),
            pl.BlockSpec((_N, 128), lambda i: (0, jnp.maximum(i - 1, 0))),
        ],
        out_specs=pl.BlockSpec((_N, 128), lambda i: (0, jnp.maximum(i - 1, 0))),
        out_shape=jax.ShapeDtypeStruct((_N, _OUT), jnp.float32),
        scratch_shapes=[pltpu.VMEM((2, _B), jnp.float32)],
    )(out2, seg2, W3, W3, g3, b3, x)

    return out
